# baseline (speedup 1.0000x reference)
import functools

import jax
import jax.numpy as jnp
from jax import lax
from jax.experimental import pallas as pl
from jax.experimental.pallas import tpu as pltpu
from jax.experimental.pallas import tpu_sc as plsc

_N_CAT = 26
_VOCAB = 100000
_EMBED = 32
_NUM_CONT = 13
_OUT_DIM = 128
_B = 16384

_NC = 2
_NS = 16
_NW = _NC * _NS
_ROWS = _N_CAT * _EMBED
_RPW = _ROWS // _NW
_CHUNK = 4096
_NCHUNK = _B // _CHUNK


def _sc_body(meta_hbm, table_hbm, out_hbm, row_v, idx_v, out_v, sem0, sem1):
    c = lax.axis_index("c")
    s = lax.axis_index("s")
    w = s * _NC + c
    sems = (sem0, sem1)
    hw = _CHUNK // 2

    def row_step(j, prev_tbl):
        g = w * _RPW + j
        tbl = g >> 5

        @pl.when(tbl != prev_tbl)
        def _():
            pltpu.sync_copy(meta_hbm.at[_NUM_CONT + tbl],
                            row_v.at[pl.ds(0, _B)])

            @plsc.parallel_loop(0, _B, 16, unroll=4)
            def conv(p):
                v = row_v[pl.ds(p, 16)]
                idx_v[pl.ds(p, 16)] = jnp.clip(
                    v.astype(jnp.int32), 0, _VOCAB - 1)

        pltpu.sync_copy(table_hbm.at[g], row_v)

        for h in range(_NCHUNK):
            buf = h % 2
            drain = pltpu.make_async_copy(
                out_v.at[buf], out_hbm.at[g, pl.ds(h * hw, hw)], sems[buf])
            if h < 2:
                @pl.when(j > 0)
                def _():
                    drain.wait()
            else:
                drain.wait()

            @plsc.parallel_loop(0, hw, 16, unroll=4)
            def gat(p, h=h, buf=buf):
                base = h * hw + p
                ie = idx_v[pl.ds(base, 16)]
                io = idx_v[pl.ds(_B // 2 + base, 16)]
                ve = plsc.load_gather(row_v, [ie])
                vo = plsc.load_gather(row_v, [io])
                pk = plsc.pack(ve, vo, format=plsc.PackFormat.INTERLEAVED)
                out_v[buf, pl.ds(p, 16)] = plsc.bitcast(pk, jnp.uint32)

            pltpu.async_copy(
                out_v.at[buf], out_hbm.at[g, pl.ds(h * hw, hw)], sems[buf])
        return tbl

    lax.fori_loop(0, _RPW, row_step, jnp.int32(-1))
    g_last = w * _RPW + (_RPW - 1)
    for buf in range(2):
        pltpu.make_async_copy(
            out_v.at[buf],
            out_hbm.at[g_last, pl.ds((2 + buf) * hw, hw)],
            sems[buf]).wait()


@functools.lru_cache(maxsize=None)
def _make_sc_gather():
    return functools.partial(
        pl.kernel,
        mesh=plsc.VectorSubcoreMesh(core_axis_name="c", subcore_axis_name="s"),
        out_type=jax.ShapeDtypeStruct((_ROWS, _B // 2), jnp.uint32),
        compiler_params=pltpu.CompilerParams(
            use_tc_tiling_on_sc=True, needs_layout_passes=False),
        scratch_types=[
            pltpu.VMEM((_VOCAB,), jnp.float32),
            pltpu.VMEM((_B,), jnp.int32),
            pltpu.VMEM((2, _CHUNK // 2), jnp.uint32),
            pltpu.SemaphoreType.DMA,
            pltpu.SemaphoreType.DMA,
        ],
    )(_sc_body)


_SQRT_HALF = 0.7071067811865476


def _gelu(x):
    return 0.5 * x * (1.0 + lax.erf(x * _SQRT_HALF))


def _lnorm0(x, g, b, eps=1e-5):
    m = jnp.mean(x, axis=0, keepdims=True)
    v = jnp.mean((x - m) * (x - m), axis=0, keepdims=True)
    return (x - m) * lax.rsqrt(v + eps) * g + b


def _mlp_subset(xn, cat, np_ln1_g, np_ln1_b, np_wT, np_b, np_ln2_g, np_ln2_b,
                g1n, b1n, g1c, b1c, w1nT, w1cT, f_b1, f_ln2_g, f_ln2_b,
                f_w2T, f_b2, f_ln3_g, f_ln3_b):
    h = _lnorm0(xn, np_ln1_g, np_ln1_b)
    h = jnp.dot(np_wT, h, preferred_element_type=jnp.float32) + np_b
    h = _gelu(h)
    xnp = _lnorm0(h, np_ln2_g, np_ln2_b)

    total = _EMBED * 2 + _N_CAT * _EMBED
    sm = jnp.sum(xnp, axis=0, keepdims=True) + jnp.sum(cat, axis=0, keepdims=True)
    m = sm / total
    dn = xnp - m
    dc = cat - m
    ss = jnp.sum(dn * dn, axis=0, keepdims=True) + jnp.sum(dc * dc, axis=0, keepdims=True)
    r = lax.rsqrt(ss / total + 1e-5)
    an = dn * r * g1n + b1n
    ac = dc * r * g1c + b1c
    y = (jnp.dot(w1nT, an, preferred_element_type=jnp.float32)
         + jnp.dot(w1cT, ac, preferred_element_type=jnp.float32)
         + f_b1)
    y = _gelu(y)
    y = _lnorm0(y, f_ln2_g, f_ln2_b)
    y = jnp.dot(f_w2T, y, preferred_element_type=jnp.float32) + f_b2
    y = _gelu(y)
    y = _lnorm0(y, f_ln3_g, f_ln3_b)
    return y.T


def _mlp_body(metaE_ref, metaO_ref, cat32_ref, *refs):
    weight_refs, (oE_ref, oO_ref) = refs[:-2], refs[-2:]
    ws = [wr[...] for wr in weight_refs]
    cat32 = cat32_ref[...]
    catE = lax.bitcast_convert_type(cat32 << jnp.uint32(16), jnp.float32)
    catO = lax.bitcast_convert_type(
        cat32 & jnp.uint32(0xFFFF0000), jnp.float32)
    oE_ref[...] = _mlp_subset(metaE_ref[0:_NUM_CONT, :], catE, *ws)
    oO_ref[...] = _mlp_subset(metaO_ref[0:_NUM_CONT, :], catO, *ws)


_BLK = 1024


def _full(shape):
    nd = len(shape)
    return pl.BlockSpec(shape, lambda i: (0,) * nd)


def kernel(meta_tensor, emb_tables, np_ln1_g, np_ln1_b, np_w, np_b, np_ln2_g,
           np_ln2_b, f_ln1_g, f_ln1_b, f_w1, f_b1, f_ln2_g, f_ln2_b, f_w2,
           f_b2, f_ln3_g, f_ln3_b):
    meta_T = meta_tensor.T
    table_T = emb_tables.transpose(0, 2, 1).reshape(_ROWS, _VOCAB)
    cat32 = _make_sc_gather()(meta_T, table_T)

    split = _EMBED * 2
    col = lambda v: v.reshape(-1, 1)
    args = (
        meta_T, meta_T, cat32,
        col(np_ln1_g), col(np_ln1_b), np_w.T, col(np_b),
        col(np_ln2_g), col(np_ln2_b),
        col(f_ln1_g[:split]), col(f_ln1_b[:split]),
        col(f_ln1_g[split:]), col(f_ln1_b[split:]),
        f_w1[:split].T, f_w1[split:].T, col(f_b1),
        col(f_ln2_g), col(f_ln2_b), f_w2.T,
        col(f_b2), col(f_ln3_g), col(f_ln3_b),
    )
    nhalf = _B // 2 // _BLK
    in_specs = [
        pl.BlockSpec((_NUM_CONT + _N_CAT, _BLK), lambda i: (0, i)),
        pl.BlockSpec((_NUM_CONT + _N_CAT, _BLK), lambda i: (0, i + nhalf)),
        pl.BlockSpec((_ROWS, _BLK), lambda i: (0, i)),
    ] + [_full(a.shape) for a in args[3:]]
    y_e, y_o = pl.pallas_call(
        _mlp_body,
        grid=(_B // 2 // _BLK,),
        in_specs=in_specs,
        out_specs=[pl.BlockSpec((_BLK, _OUT_DIM), lambda i: (i, 0)),
                   pl.BlockSpec((_BLK, _OUT_DIM), lambda i: (i, 0))],
        out_shape=[jax.ShapeDtypeStruct((_B // 2, _OUT_DIM), jnp.float32),
                   jax.ShapeDtypeStruct((_B // 2, _OUT_DIM), jnp.float32)],
        compiler_params=pltpu.CompilerParams(
            dimension_semantics=("parallel",),
        ),
    )(*args)
    return jnp.concatenate([y_e, y_o], axis=0)


# speedup vs baseline: 1.0095x; 1.0095x over previous
import functools

import jax
import jax.numpy as jnp
from jax import lax
from jax.experimental import pallas as pl
from jax.experimental.pallas import tpu as pltpu
from jax.experimental.pallas import tpu_sc as plsc

_N_CAT = 26
_VOCAB = 100000
_EMBED = 32
_NUM_CONT = 13
_OUT_DIM = 128
_B = 16384

_NC = 2
_NS = 16
_NW = _NC * _NS
_ROWS = _N_CAT * _EMBED
_RPW = _ROWS // _NW
_CHUNK = 4096
_NCHUNK = _B // _CHUNK


def _sc_body(meta_hbm, table_hbm, out_hbm, row_v, idx_v, out_v, sem0, sem1):
    c = lax.axis_index("c")
    s = lax.axis_index("s")
    w = s * _NC + c
    sems = (sem0, sem1)
    hw = _CHUNK // 2

    def row_step(j, prev_tbl):
        g = w * _RPW + j
        tbl = g >> 5

        @pl.when(tbl != prev_tbl)
        def _():
            pltpu.sync_copy(meta_hbm.at[_NUM_CONT + tbl],
                            row_v.at[pl.ds(0, _B)])

            @plsc.parallel_loop(0, _B, 16, unroll=4)
            def conv(p):
                v = row_v[pl.ds(p, 16)]
                idx_v[pl.ds(p, 16)] = jnp.clip(
                    v.astype(jnp.int32), 0, _VOCAB - 1)

        pltpu.sync_copy(table_hbm.at[g], row_v)

        for h in range(_NCHUNK):
            buf = h % 2
            drain = pltpu.make_async_copy(
                out_v.at[buf], out_hbm.at[g, pl.ds(h * hw, hw)], sems[buf])
            if h < 2:
                @pl.when(j > 0)
                def _():
                    drain.wait()
            else:
                drain.wait()

            @plsc.parallel_loop(0, hw, 16, unroll=8)
            def gat(p, h=h, buf=buf):
                base = h * hw + p
                ie = idx_v[pl.ds(base, 16)]
                io = idx_v[pl.ds(_B // 2 + base, 16)]
                ve = plsc.load_gather(row_v, [ie])
                vo = plsc.load_gather(row_v, [io])
                pk = plsc.pack(ve, vo, format=plsc.PackFormat.INTERLEAVED)
                out_v[buf, pl.ds(p, 16)] = plsc.bitcast(pk, jnp.uint32)

            pltpu.async_copy(
                out_v.at[buf], out_hbm.at[g, pl.ds(h * hw, hw)], sems[buf])
        return tbl

    lax.fori_loop(0, _RPW, row_step, jnp.int32(-1))
    g_last = w * _RPW + (_RPW - 1)
    for buf in range(2):
        pltpu.make_async_copy(
            out_v.at[buf],
            out_hbm.at[g_last, pl.ds((2 + buf) * hw, hw)],
            sems[buf]).wait()


@functools.lru_cache(maxsize=None)
def _make_sc_gather():
    return functools.partial(
        pl.kernel,
        mesh=plsc.VectorSubcoreMesh(core_axis_name="c", subcore_axis_name="s"),
        out_type=jax.ShapeDtypeStruct((_ROWS, _B // 2), jnp.uint32),
        compiler_params=pltpu.CompilerParams(
            use_tc_tiling_on_sc=True, needs_layout_passes=False),
        scratch_types=[
            pltpu.VMEM((_VOCAB,), jnp.float32),
            pltpu.VMEM((_B,), jnp.int32),
            pltpu.VMEM((2, _CHUNK // 2), jnp.uint32),
            pltpu.SemaphoreType.DMA,
            pltpu.SemaphoreType.DMA,
        ],
    )(_sc_body)


_SQRT_HALF = 0.7071067811865476


def _gelu(x):
    return 0.5 * x * (1.0 + lax.erf(x * _SQRT_HALF))


def _lnorm0(x, g, b, eps=1e-5):
    m = jnp.mean(x, axis=0, keepdims=True)
    v = jnp.mean((x - m) * (x - m), axis=0, keepdims=True)
    return (x - m) * lax.rsqrt(v + eps) * g + b


def _mlp_subset(xn, cat, np_ln1_g, np_ln1_b, np_wT, np_b, np_ln2_g, np_ln2_b,
                g1n, b1n, g1c, b1c, w1nT, w1cT, f_b1, f_ln2_g, f_ln2_b,
                f_w2T, f_b2, f_ln3_g, f_ln3_b):
    h = _lnorm0(xn, np_ln1_g, np_ln1_b)
    h = jnp.dot(np_wT, h, preferred_element_type=jnp.float32) + np_b
    h = _gelu(h)
    xnp = _lnorm0(h, np_ln2_g, np_ln2_b)

    total = _EMBED * 2 + _N_CAT * _EMBED
    sm = jnp.sum(xnp, axis=0, keepdims=True) + jnp.sum(cat, axis=0, keepdims=True)
    m = sm / total
    dn = xnp - m
    dc = cat - m
    ss = jnp.sum(dn * dn, axis=0, keepdims=True) + jnp.sum(dc * dc, axis=0, keepdims=True)
    r = lax.rsqrt(ss / total + 1e-5)
    an = dn * r * g1n + b1n
    ac = dc * r * g1c + b1c
    y = (jnp.dot(w1nT, an, preferred_element_type=jnp.float32)
         + jnp.dot(w1cT, ac, preferred_element_type=jnp.float32)
         + f_b1)
    y = _gelu(y)
    y = _lnorm0(y, f_ln2_g, f_ln2_b)
    y = jnp.dot(f_w2T, y, preferred_element_type=jnp.float32) + f_b2
    y = _gelu(y)
    y = _lnorm0(y, f_ln3_g, f_ln3_b)
    return y.T


def _mlp_body(metaE_ref, metaO_ref, cat32_ref, *refs):
    weight_refs, (oE_ref, oO_ref) = refs[:-2], refs[-2:]
    ws = [wr[...] for wr in weight_refs]
    cat32 = cat32_ref[...]
    catE = lax.bitcast_convert_type(cat32 << jnp.uint32(16), jnp.float32)
    catO = lax.bitcast_convert_type(
        cat32 & jnp.uint32(0xFFFF0000), jnp.float32)
    oE_ref[...] = _mlp_subset(metaE_ref[0:_NUM_CONT, :], catE, *ws)
    oO_ref[...] = _mlp_subset(metaO_ref[0:_NUM_CONT, :], catO, *ws)


_BLK = 2048


def _full(shape):
    nd = len(shape)
    return pl.BlockSpec(shape, lambda i: (0,) * nd)


def kernel(meta_tensor, emb_tables, np_ln1_g, np_ln1_b, np_w, np_b, np_ln2_g,
           np_ln2_b, f_ln1_g, f_ln1_b, f_w1, f_b1, f_ln2_g, f_ln2_b, f_w2,
           f_b2, f_ln3_g, f_ln3_b):
    meta_T = meta_tensor.T
    table_T = emb_tables.transpose(0, 2, 1).reshape(_ROWS, _VOCAB)
    cat32 = _make_sc_gather()(meta_T, table_T)

    split = _EMBED * 2
    col = lambda v: v.reshape(-1, 1)
    args = (
        meta_T, meta_T, cat32,
        col(np_ln1_g), col(np_ln1_b), np_w.T, col(np_b),
        col(np_ln2_g), col(np_ln2_b),
        col(f_ln1_g[:split]), col(f_ln1_b[:split]),
        col(f_ln1_g[split:]), col(f_ln1_b[split:]),
        f_w1[:split].T, f_w1[split:].T, col(f_b1),
        col(f_ln2_g), col(f_ln2_b), f_w2.T,
        col(f_b2), col(f_ln3_g), col(f_ln3_b),
    )
    nhalf = _B // 2 // _BLK
    in_specs = [
        pl.BlockSpec((_NUM_CONT + _N_CAT, _BLK), lambda i: (0, i)),
        pl.BlockSpec((_NUM_CONT + _N_CAT, _BLK), lambda i: (0, i + nhalf)),
        pl.BlockSpec((_ROWS, _BLK), lambda i: (0, i)),
    ] + [_full(a.shape) for a in args[3:]]
    y_e, y_o = pl.pallas_call(
        _mlp_body,
        grid=(_B // 2 // _BLK,),
        in_specs=in_specs,
        out_specs=[pl.BlockSpec((_BLK, _OUT_DIM), lambda i: (i, 0)),
                   pl.BlockSpec((_BLK, _OUT_DIM), lambda i: (i, 0))],
        out_shape=[jax.ShapeDtypeStruct((_B // 2, _OUT_DIM), jnp.float32),
                   jax.ShapeDtypeStruct((_B // 2, _OUT_DIM), jnp.float32)],
        compiler_params=pltpu.CompilerParams(
            dimension_semantics=("parallel",),
        ),
    )(*args)
    return jnp.concatenate([y_e, y_o], axis=0)


# speedup vs baseline: 1.0126x; 1.0031x over previous
import functools

import jax
import jax.numpy as jnp
from jax import lax
from jax.experimental import pallas as pl
from jax.experimental.pallas import tpu as pltpu
from jax.experimental.pallas import tpu_sc as plsc

_N_CAT = 26
_VOCAB = 100000
_EMBED = 32
_NUM_CONT = 13
_OUT_DIM = 128
_B = 16384

_NC = 2
_NS = 16
_NW = _NC * _NS
_ROWS = _N_CAT * _EMBED
_RPW = _ROWS // _NW
_CHUNK = 4096
_NCHUNK = _B // _CHUNK


def _sc_body(meta_hbm, table_hbm, out_hbm, row_v, idx_v, out_v, sem0, sem1,
             semr):
    c = lax.axis_index("c")
    s = lax.axis_index("s")
    w = s * _NC + c
    sems = (sem0, sem1)
    hw = _CHUNK // 2

    def row_step(j, prev_tbl):
        g = w * _RPW + j
        tbl = g >> 5

        row_cp = pltpu.async_copy(table_hbm.at[g], row_v, semr)

        @pl.when(tbl != prev_tbl)
        def _():
            pltpu.sync_copy(meta_hbm.at[_NUM_CONT + tbl], idx_v)

            @plsc.parallel_loop(0, _B, 16, unroll=4)
            def conv(p):
                v = idx_v[pl.ds(p, 16)]
                ii = jnp.clip(v.astype(jnp.int32), 0, _VOCAB - 1)
                idx_v[pl.ds(p, 16)] = plsc.bitcast(ii, jnp.float32)

        row_cp.wait()

        for h in range(_NCHUNK):
            buf = h % 2
            drain = pltpu.make_async_copy(
                out_v.at[buf], out_hbm.at[g, pl.ds(h * hw, hw)], sems[buf])
            if h < 2:
                @pl.when(j > 0)
                def _():
                    drain.wait()
            else:
                drain.wait()

            @plsc.parallel_loop(0, hw, 16, unroll=8)
            def gat(p, h=h, buf=buf):
                base = h * hw + p
                ie = plsc.bitcast(idx_v[pl.ds(base, 16)], jnp.int32)
                io = plsc.bitcast(idx_v[pl.ds(_B // 2 + base, 16)], jnp.int32)
                ve = plsc.load_gather(row_v, [ie])
                vo = plsc.load_gather(row_v, [io])
                pk = plsc.pack(ve, vo, format=plsc.PackFormat.INTERLEAVED)
                out_v[buf, pl.ds(p, 16)] = plsc.bitcast(pk, jnp.uint32)

            pltpu.async_copy(
                out_v.at[buf], out_hbm.at[g, pl.ds(h * hw, hw)], sems[buf])
        return tbl

    lax.fori_loop(0, _RPW, row_step, jnp.int32(-1))
    g_last = w * _RPW + (_RPW - 1)
    for buf in range(2):
        pltpu.make_async_copy(
            out_v.at[buf],
            out_hbm.at[g_last, pl.ds((2 + buf) * hw, hw)],
            sems[buf]).wait()


@functools.lru_cache(maxsize=None)
def _make_sc_gather():
    return functools.partial(
        pl.kernel,
        mesh=plsc.VectorSubcoreMesh(core_axis_name="c", subcore_axis_name="s"),
        out_type=jax.ShapeDtypeStruct((_ROWS, _B // 2), jnp.uint32),
        compiler_params=pltpu.CompilerParams(
            use_tc_tiling_on_sc=True, needs_layout_passes=False),
        scratch_types=[
            pltpu.VMEM((_VOCAB,), jnp.float32),
            pltpu.VMEM((_B,), jnp.float32),
            pltpu.VMEM((2, _CHUNK // 2), jnp.uint32),
            pltpu.SemaphoreType.DMA,
            pltpu.SemaphoreType.DMA,
            pltpu.SemaphoreType.DMA,
        ],
    )(_sc_body)


_SQRT_HALF = 0.7071067811865476


def _gelu(x):
    return 0.5 * x * (1.0 + lax.erf(x * _SQRT_HALF))


def _lnorm0(x, g, b, eps=1e-5):
    m = jnp.mean(x, axis=0, keepdims=True)
    v = jnp.mean((x - m) * (x - m), axis=0, keepdims=True)
    return (x - m) * lax.rsqrt(v + eps) * g + b


def _mlp_subset(xn, cat, np_ln1_g, np_ln1_b, np_wT, np_b, np_ln2_g, np_ln2_b,
                g1n, b1n, g1c, b1c, w1nT, w1cT, f_b1, f_ln2_g, f_ln2_b,
                f_w2T, f_b2, f_ln3_g, f_ln3_b):
    h = _lnorm0(xn, np_ln1_g, np_ln1_b)
    h = jnp.dot(np_wT, h, preferred_element_type=jnp.float32) + np_b
    h = _gelu(h)
    xnp = _lnorm0(h, np_ln2_g, np_ln2_b)

    total = _EMBED * 2 + _N_CAT * _EMBED
    sm = jnp.sum(xnp, axis=0, keepdims=True) + jnp.sum(cat, axis=0, keepdims=True)
    m = sm / total
    dn = xnp - m
    dc = cat - m
    ss = jnp.sum(dn * dn, axis=0, keepdims=True) + jnp.sum(dc * dc, axis=0, keepdims=True)
    r = lax.rsqrt(ss / total + 1e-5)
    an = dn * r * g1n + b1n
    ac = dc * r * g1c + b1c
    y = (jnp.dot(w1nT, an, preferred_element_type=jnp.float32)
         + jnp.dot(w1cT, ac, preferred_element_type=jnp.float32)
         + f_b1)
    y = _gelu(y)
    y = _lnorm0(y, f_ln2_g, f_ln2_b)
    y = jnp.dot(f_w2T, y, preferred_element_type=jnp.float32) + f_b2
    y = _gelu(y)
    y = _lnorm0(y, f_ln3_g, f_ln3_b)
    return y.T


def _mlp_body(metaE_ref, metaO_ref, cat32_ref, *refs):
    weight_refs, (oE_ref, oO_ref) = refs[:-2], refs[-2:]
    ws = [wr[...] for wr in weight_refs]
    cat32 = cat32_ref[...]
    catE = lax.bitcast_convert_type(cat32 << jnp.uint32(16), jnp.float32)
    catO = lax.bitcast_convert_type(
        cat32 & jnp.uint32(0xFFFF0000), jnp.float32)
    oE_ref[...] = _mlp_subset(metaE_ref[0:_NUM_CONT, :], catE, *ws)
    oO_ref[...] = _mlp_subset(metaO_ref[0:_NUM_CONT, :], catO, *ws)


_BLK = 2048


def _full(shape):
    nd = len(shape)
    return pl.BlockSpec(shape, lambda i: (0,) * nd)


def kernel(meta_tensor, emb_tables, np_ln1_g, np_ln1_b, np_w, np_b, np_ln2_g,
           np_ln2_b, f_ln1_g, f_ln1_b, f_w1, f_b1, f_ln2_g, f_ln2_b, f_w2,
           f_b2, f_ln3_g, f_ln3_b):
    meta_T = meta_tensor.T
    table_T = emb_tables.transpose(0, 2, 1).reshape(_ROWS, _VOCAB)
    cat32 = _make_sc_gather()(meta_T, table_T)

    split = _EMBED * 2
    col = lambda v: v.reshape(-1, 1)
    args = (
        meta_T, meta_T, cat32,
        col(np_ln1_g), col(np_ln1_b), np_w.T, col(np_b),
        col(np_ln2_g), col(np_ln2_b),
        col(f_ln1_g[:split]), col(f_ln1_b[:split]),
        col(f_ln1_g[split:]), col(f_ln1_b[split:]),
        f_w1[:split].T, f_w1[split:].T, col(f_b1),
        col(f_ln2_g), col(f_ln2_b), f_w2.T,
        col(f_b2), col(f_ln3_g), col(f_ln3_b),
    )
    nhalf = _B // 2 // _BLK
    in_specs = [
        pl.BlockSpec((_NUM_CONT + _N_CAT, _BLK), lambda i: (0, i)),
        pl.BlockSpec((_NUM_CONT + _N_CAT, _BLK), lambda i: (0, i + nhalf)),
        pl.BlockSpec((_ROWS, _BLK), lambda i: (0, i)),
    ] + [_full(a.shape) for a in args[3:]]
    y_e, y_o = pl.pallas_call(
        _mlp_body,
        grid=(_B // 2 // _BLK,),
        in_specs=in_specs,
        out_specs=[pl.BlockSpec((_BLK, _OUT_DIM), lambda i: (i, 0)),
                   pl.BlockSpec((_BLK, _OUT_DIM), lambda i: (i, 0))],
        out_shape=[jax.ShapeDtypeStruct((_B // 2, _OUT_DIM), jnp.float32),
                   jax.ShapeDtypeStruct((_B // 2, _OUT_DIM), jnp.float32)],
        compiler_params=pltpu.CompilerParams(
            dimension_semantics=("parallel",),
        ),
    )(*args)
    return jnp.concatenate([y_e, y_o], axis=0)
